# baseline (speedup 1.0000x reference)
import jax
import jax.numpy as jnp
from jax import lax
from jax.experimental import pallas as pl
from jax.experimental.pallas import tpu as pltpu

_CODE = 16
_CH = 2048
_NBUF = 2
_KM = 1152


def _vq_body(x_hbm, wet_ref, wl_ref, wdt_ref, y_hbm, xb, xlb, yb,
             in_sem, inl_sem, out_sem):
    n = x_hbm.shape[0] // _CH
    wet = wet_ref[...]
    wl = wl_ref[...]
    wdt = wdt_ref[...]

    def start_in(i, slot):
        pltpu.make_async_copy(
            x_hbm.at[pl.ds(i * _CH, _CH), pl.ds(0, _KM)], xb.at[slot],
            in_sem.at[slot]
        ).start()
        pltpu.make_async_copy(
            x_hbm.at[pl.ds(i * _CH, _CH), pl.ds(_KM, 1)], xlb.at[slot],
            inl_sem.at[slot]
        ).start()

    def wait_in(slot):
        pltpu.make_async_copy(
            x_hbm.at[pl.ds(0, _CH), pl.ds(0, _KM)], xb.at[slot],
            in_sem.at[slot]
        ).wait()
        pltpu.make_async_copy(
            x_hbm.at[pl.ds(0, _CH), pl.ds(_KM, 1)], xlb.at[slot],
            inl_sem.at[slot]
        ).wait()

    def start_out(i, slot):
        pltpu.make_async_copy(
            yb.at[slot], y_hbm.at[pl.ds(i * _CH, _CH)], out_sem.at[slot]
        ).start()

    def wait_out(slot):
        pltpu.make_async_copy(
            yb.at[slot], y_hbm.at[pl.ds(0, _CH)], out_sem.at[slot]
        ).wait()

    for s in range(_NBUF):
        start_in(s, s)

    for i in range(n):
        slot = i % _NBUF
        wait_in(slot)
        x = xb[slot]
        xl = xlb[slot]
        h = lax.dot_general(x, wet, (((1,), (0,)), ((), ())),
                            preferred_element_type=jnp.float32)
        h = h + xl * wl
        mx = jnp.max(h, axis=1, keepdims=True)
        iota = lax.broadcasted_iota(jnp.int32, h.shape, 1)
        first = jnp.min(jnp.where(h >= mx, iota, _CODE), axis=1, keepdims=True)
        onehot = (iota == first).astype(jnp.float32)
        if i + _NBUF < n:
            start_in(i + _NBUF, slot)
        if i >= _NBUF:
            wait_out(slot)
        yb[slot] = lax.dot_general(onehot, wdt, (((1,), (0,)), ((), ())),
                                   preferred_element_type=jnp.float32)
        start_out(i, slot)

    for i in range(max(n - _NBUF, 0), n):
        wait_out(i % _NBUF)


def kernel(x, W_enc, W_dec):
    B, IN = x.shape
    OUT = W_dec.shape[0]
    wet = W_enc.T
    return pl.pallas_call(
        _vq_body,
        in_specs=[
            pl.BlockSpec(memory_space=pl.ANY),
            pl.BlockSpec(memory_space=pltpu.VMEM),
            pl.BlockSpec(memory_space=pltpu.VMEM),
            pl.BlockSpec(memory_space=pltpu.VMEM),
        ],
        out_specs=pl.BlockSpec(memory_space=pl.ANY),
        out_shape=jax.ShapeDtypeStruct((B, OUT), jnp.float32),
        scratch_shapes=[
            pltpu.VMEM((_NBUF, _CH, _KM), jnp.float32),
            pltpu.VMEM((_NBUF, _CH, 1), jnp.float32),
            pltpu.VMEM((_NBUF, _CH, OUT), jnp.float32),
            pltpu.SemaphoreType.DMA((_NBUF,)),
            pltpu.SemaphoreType.DMA((_NBUF,)),
            pltpu.SemaphoreType.DMA((_NBUF,)),
        ],
        compiler_params=pltpu.CompilerParams(
            vmem_limit_bytes=100 * 1024 * 1024,
        ),
    )(x, wet[:_KM], wet[_KM:], W_dec.T)
